# baseline (speedup 1.0000x reference)
import math

import jax
import jax.numpy as jnp
from jax.experimental import pallas as pl
from jax.experimental.pallas import tpu as pltpu

_B, _C, _H, _W = 16, 96, 224, 224
_HW = _H * _W
_NCG = 4
_CG = _C // _NCG
_K_INACTIVE = math.ceil(0.5 * _C)


def _pool_gate_kernel(x0_ref, x1_ref, x2_ref, x3_ref, m_ref, lasso_ref,
                      w_ref, b_ref, out_ref, lasso_out_ref, sacc_ref):
    i = pl.program_id(0)

    m_row = m_ref[0]
    parts = []
    for x_ref in (x0_ref, x1_ref, x2_ref, x3_ref):
        xm = x_ref[0] * m_row
        parts.append(jnp.sum(xm.reshape(_CG, _HW // 128, 128), axis=1))
    acc = jnp.concatenate(parts, axis=0)
    active = jnp.sum(m_row)

    @pl.when(i == 0)
    def _init_lasso_acc():
        sacc_ref[0] = 0.0

    ii = jax.lax.broadcasted_iota(jnp.int32, (_C, _C), 0)
    jj = jax.lax.broadcasted_iota(jnp.int32, (_C, _C), 1)
    eye = (ii == jj).astype(jnp.float32)
    pooled_col = jnp.sum(acc, axis=1, keepdims=True) / active
    pooled_row = jax.lax.dot_general(
        pooled_col, eye, (((0,), (0,)), ((), ())),
        preferred_element_type=jnp.float32,
        precision=jax.lax.Precision.HIGHEST)
    logits = jax.lax.dot_general(
        pooled_row, w_ref[...], (((1,), (1,)), ((), ())),
        preferred_element_type=jnp.float32,
        precision=jax.lax.Precision.DEFAULT)
    s_row = jax.nn.sigmoid(logits + b_ref[...])
    s_col = jax.lax.dot_general(
        eye, s_row, (((1,), (1,)), ((), ())),
        preferred_element_type=jnp.float32,
        precision=jax.lax.Precision.HIGHEST)
    s_r = jnp.broadcast_to(s_col, (_C, _C))
    s_c = jnp.broadcast_to(s_row, (_C, _C))
    beats = (s_r < s_c) | ((s_r == s_c) & (ii < jj))
    rank = jnp.sum(beats.astype(jnp.int32), axis=0, keepdims=True)
    out_ref[pl.ds(i, 1), :] = (rank >= _K_INACTIVE).astype(jnp.int32)
    sacc_ref[0] += jnp.sum(s_row)

    @pl.when(i == _B - 1)
    def _final():
        lasso_out_ref[0, 0] = lasso_ref[0, 0] + sacc_ref[0] / _B


def kernel(x, masked_feat, lasso_sum, W, b):
    xr = x.reshape(_B, _C, _HW)
    mr = masked_feat.reshape(_B, 1, _HW)
    lr = lasso_sum.reshape(1, 1)
    br = b.reshape(1, _C)

    def xspec(k):
        return pl.BlockSpec((1, _CG, _HW), lambda i, k=k: (i, k, 0))

    out, lasso = pl.pallas_call(
        _pool_gate_kernel,
        grid=(_B,),
        in_specs=[
            xspec(0), xspec(1), xspec(2), xspec(3),
            pl.BlockSpec((1, 1, _HW), lambda i: (i, 0, 0)),
            pl.BlockSpec(memory_space=pltpu.SMEM),
            pl.BlockSpec((_C, _C), lambda i: (0, 0)),
            pl.BlockSpec((1, _C), lambda i: (0, 0)),
        ],
        out_specs=[
            pl.BlockSpec((_B, _C), lambda i: (0, 0)),
            pl.BlockSpec(memory_space=pltpu.SMEM),
        ],
        out_shape=[
            jax.ShapeDtypeStruct((_B, _C), jnp.int32),
            jax.ShapeDtypeStruct((1, 1), jnp.float32),
        ],
        scratch_shapes=[
            pltpu.SMEM((1,), jnp.float32),
        ],
    )(xr, xr, xr, xr, mr, lr, W, br)
    return out, lasso.reshape(())


# speedup vs baseline: 1.0046x; 1.0046x over previous
import math

import jax
import jax.numpy as jnp
from jax.experimental import pallas as pl
from jax.experimental.pallas import tpu as pltpu

_B, _C, _H, _W = 16, 96, 224, 224
_HW = _H * _W
_NS = 4
_CG = _C // _NS
_K_INACTIVE = math.ceil(0.5 * _C)


def _pool_gate_kernel(x_hbm, m_ref, lasso_ref, w_ref, b_ref,
                      out_ref, lasso_out_ref, xbuf, sems, sacc_ref):
    i = pl.program_id(0)

    def copies(slot, b):
        return [
            pltpu.make_async_copy(
                x_hbm.at[b, pl.ds(s * _CG, _CG), :],
                xbuf.at[slot, s],
                sems.at[slot, s])
            for s in range(_NS)
        ]

    @pl.when(i == 0)
    def _prime():
        for c in copies(0, 0):
            c.start()

    @pl.when(i + 1 < _B)
    def _prefetch():
        for c in copies((i + 1) % 2, i + 1):
            c.start()

    slot = i % 2
    for c in copies(slot, i):
        c.wait()

    m_row = m_ref[0]
    parts = []
    for s in range(_NS):
        xm = xbuf[slot, s] * m_row
        parts.append(jnp.sum(xm.reshape(_CG, _HW // 128, 128), axis=1))
    acc = jnp.concatenate(parts, axis=0)
    active = jnp.sum(m_row)

    @pl.when(i == 0)
    def _init_lasso_acc():
        sacc_ref[0] = 0.0

    ii = jax.lax.broadcasted_iota(jnp.int32, (_C, _C), 0)
    jj = jax.lax.broadcasted_iota(jnp.int32, (_C, _C), 1)
    eye = (ii == jj).astype(jnp.float32)
    pooled_col = jnp.sum(acc, axis=1, keepdims=True) / active
    pooled_row = jax.lax.dot_general(
        pooled_col, eye, (((0,), (0,)), ((), ())),
        preferred_element_type=jnp.float32,
        precision=jax.lax.Precision.HIGHEST)
    logits = jax.lax.dot_general(
        pooled_row, w_ref[...], (((1,), (1,)), ((), ())),
        preferred_element_type=jnp.float32,
        precision=jax.lax.Precision.DEFAULT)
    s_row = jax.nn.sigmoid(logits + b_ref[...])
    s_col = jax.lax.dot_general(
        eye, s_row, (((1,), (1,)), ((), ())),
        preferred_element_type=jnp.float32,
        precision=jax.lax.Precision.HIGHEST)
    s_r = jnp.broadcast_to(s_col, (_C, _C))
    s_c = jnp.broadcast_to(s_row, (_C, _C))
    beats = (s_r < s_c) | ((s_r == s_c) & (ii < jj))
    rank = jnp.sum(beats.astype(jnp.int32), axis=0, keepdims=True)
    out_ref[pl.ds(i, 1), :] = (rank >= _K_INACTIVE).astype(jnp.int32)
    sacc_ref[0] += jnp.sum(s_row)

    @pl.when(i == _B - 1)
    def _final():
        lasso_out_ref[0, 0] = lasso_ref[0, 0] + sacc_ref[0] / _B


def kernel(x, masked_feat, lasso_sum, W, b):
    xr = x.reshape(_B, _C, _HW)
    mr = masked_feat.reshape(_B, 1, _HW)
    lr = lasso_sum.reshape(1, 1)
    br = b.reshape(1, _C)

    out, lasso = pl.pallas_call(
        _pool_gate_kernel,
        grid=(_B,),
        in_specs=[
            pl.BlockSpec(memory_space=pl.ANY),
            pl.BlockSpec((1, 1, _HW), lambda i: (i, 0, 0)),
            pl.BlockSpec(memory_space=pltpu.SMEM),
            pl.BlockSpec((_C, _C), lambda i: (0, 0)),
            pl.BlockSpec((1, _C), lambda i: (0, 0)),
        ],
        out_specs=[
            pl.BlockSpec((_B, _C), lambda i: (0, 0)),
            pl.BlockSpec(memory_space=pltpu.SMEM),
        ],
        out_shape=[
            jax.ShapeDtypeStruct((_B, _C), jnp.int32),
            jax.ShapeDtypeStruct((1, 1), jnp.float32),
        ],
        scratch_shapes=[
            pltpu.VMEM((2, _NS, _CG, _HW), jnp.float32),
            pltpu.SemaphoreType.DMA((2, _NS)),
            pltpu.SMEM((1,), jnp.float32),
        ],
    )(xr, mr, lr, W, br)
    return out, lasso.reshape(())
